# Initial kernel scaffold; baseline (speedup 1.0000x reference)
import functools

import numpy as np
import jax
import jax.numpy as jnp
from jax import lax
from jax.experimental import pallas as pl
from jax.experimental.pallas import tpu as pltpu
from jax.experimental.pallas import tpu_sc as plsc

NUM_LAYER = 3
EMB = 128
N_NODES = 10000
N_EDGES = 320000
ATOM_FEATS = 9
ATOM_VOCAB = 120
BOND_FEATS = 3
BOND_VOCAB = 6

NC = 2
NS = 16
LANES = 16
NW = NC * NS

CHUNK = 128
CPW = -(-N_EDGES // (NW * CHUNK))
EPW = CPW * CHUNK
EP = NW * EPW

ROWS_PER_TILE = 626
NPAD = NS * ROWS_PER_TILE
DUMMY = N_NODES
_TILE_CHUNKS = [(0, 128), (128, 128), (256, 128), (384, 128), (512, 114)]

NCODE = BOND_VOCAB ** BOND_FEATS

_MM = dict(preferred_element_type=jnp.float32, precision=lax.Precision.HIGHEST)


def _sc_msg_body(h_hbm, src_hbm, dst_hbm, code_hbm, tbl_hbm, out_hbm,
                 src_v, dst_v, code_v, hbuf, ebuf, acc_sh):
    c = lax.axis_index("c")
    s = lax.axis_index("s")
    wid = c * NS + s

    pltpu.sync_copy(src_hbm.at[wid], src_v)
    pltpu.sync_copy(dst_hbm.at[wid], dst_v)
    pltpu.sync_copy(code_hbm.at[wid], code_v)

    @pl.loop(0, CHUNK)
    def _zero(r):
        for k in range(EMB // LANES):
            hbuf[r, pl.ds(k * LANES, LANES)] = jnp.zeros((LANES,), jnp.float32)

    r0 = s * ROWS_PER_TILE
    for off, n in _TILE_CHUNKS:
        pltpu.sync_copy(hbuf.at[pl.ds(0, n)], acc_sh.at[pl.ds(r0 + off, n)])
    plsc.subcore_barrier()

    @pl.loop(0, CPW)
    def _chunk(j):
        pltpu.sync_copy(h_hbm.at[src_v.at[j]], hbuf)
        pltpu.sync_copy(tbl_hbm.at[code_v.at[j]], ebuf)

        @pl.loop(0, CHUNK)
        def _row(r):
            for k in range(EMB // LANES):
                sl = (r, pl.ds(k * LANES, LANES))
                hbuf[sl] = jnp.maximum(hbuf[sl] + ebuf[sl], 0.0)

        pltpu.sync_copy(hbuf, acc_sh.at[dst_v.at[j]], add=True)

    plsc.subcore_barrier()
    for off, n in _TILE_CHUNKS:
        pltpu.sync_copy(acc_sh.at[pl.ds(r0 + off, n)], hbuf.at[pl.ds(0, n)])
        pltpu.sync_copy(hbuf.at[pl.ds(0, n)], out_hbm.at[c, pl.ds(r0 + off, n)])


_sc_msg = pl.kernel(
    _sc_msg_body,
    out_type=jax.ShapeDtypeStruct((NC, NPAD, EMB), jnp.float32),
    mesh=plsc.VectorSubcoreMesh(core_axis_name="c", subcore_axis_name="s"),
    scratch_types=[
        pltpu.VMEM((CPW, CHUNK), jnp.int32),
        pltpu.VMEM((CPW, CHUNK), jnp.int32),
        pltpu.VMEM((CPW, CHUNK), jnp.int32),
        pltpu.VMEM((CHUNK, EMB), jnp.float32),
        pltpu.VMEM((CHUNK, EMB), jnp.float32),
        pltpu.VMEM_SHARED((NPAD, EMB), jnp.float32),
    ],
)


def _prep_body(x_ref, atab_ref, btab_ref, h_ref, t_ref):
    xv = x_ref[...]
    h = jnp.zeros((N_NODES, EMB), jnp.float32)
    iota_a = lax.broadcasted_iota(jnp.int32, (1, ATOM_VOCAB), 1)
    for col in range(ATOM_FEATS):
        oh = (xv[:, col:col + 1] == iota_a).astype(jnp.float32)
        h = h + jnp.dot(oh, atab_ref[col], **_MM)
    h_ref[...] = h

    codes = np.arange(NCODE)
    digs = [codes // 36, (codes // 6) % 6, codes % 6]
    for l in range(NUM_LAYER):
        t = jnp.zeros((NCODE, EMB), jnp.float32)
        for col in range(BOND_FEATS):
            oh = jnp.asarray(
                (digs[col][:, None] == np.arange(BOND_VOCAB)[None, :])
                .astype(np.float32))
            t = t + jnp.dot(oh, btab_ref[l, col], **_MM)
        t_ref[l] = t


_prep = pl.pallas_call(
    _prep_body,
    out_shape=[
        jax.ShapeDtypeStruct((N_NODES, EMB), jnp.float32),
        jax.ShapeDtypeStruct((NUM_LAYER, NCODE, EMB), jnp.float32),
    ],
)


def _bn(z, g, b):
    m = jnp.mean(z, axis=0, keepdims=True)
    zc = z - m
    v = jnp.mean(zc * zc, axis=0, keepdims=True)
    return zc * lax.rsqrt(v + 1e-5) * g + b


def _mlp_body(h_ref, p_ref, w1_ref, b1_ref, g1_ref, be1_ref, w2_ref, b2_ref,
              bng_ref, bnb_ref, eps_ref, o_ref, *, final_relu):
    h = h_ref[...]
    aggr = p_ref[0, :N_NODES, :] + p_ref[1, :N_NODES, :]
    z = (1.0 + eps_ref[0]) * h + aggr
    y = jnp.dot(z, w1_ref[...], **_MM) + b1_ref[...]
    y = _bn(y, g1_ref[...], be1_ref[...])
    y = jnp.maximum(y, 0.0)
    z2 = jnp.dot(y, w2_ref[...], **_MM) + b2_ref[...]
    z2 = _bn(z2, bng_ref[...], bnb_ref[...])
    if final_relu:
        z2 = jnp.maximum(z2, 0.0)
    o_ref[...] = z2


def _make_mlp(final_relu):
    return pl.pallas_call(
        functools.partial(_mlp_body, final_relu=final_relu),
        out_shape=jax.ShapeDtypeStruct((N_NODES, EMB), jnp.float32),
        in_specs=[pl.BlockSpec(memory_space=pltpu.VMEM)] * 10
        + [pl.BlockSpec(memory_space=pltpu.SMEM)],
    )


_mlp_mid = _make_mlp(True)
_mlp_last = _make_mlp(False)


def kernel(x, edge_index, edge_attr, atom_tab, bond_tab,
           W1, b1, g1, be1, W2, b2, eps_p, bng, bnb):
    x = x.astype(jnp.int32)
    src = edge_index[0].astype(jnp.int32)
    dst = edge_index[1].astype(jnp.int32)
    ea = edge_attr.astype(jnp.int32)
    code = ea[:, 0] * 36 + ea[:, 1] * 6 + ea[:, 2]

    pad = EP - N_EDGES
    src_t = jnp.concatenate([src, jnp.zeros((pad,), jnp.int32)])
    dst_t = jnp.concatenate([dst, jnp.full((pad,), DUMMY, jnp.int32)])
    code_t = jnp.concatenate([code, jnp.zeros((pad,), jnp.int32)])
    src_t = src_t.reshape(NW, CPW, CHUNK)
    dst_t = dst_t.reshape(NW, CPW, CHUNK)
    code_t = code_t.reshape(NW, CPW, CHUNK)

    h, tbl = _prep(x, atom_tab, bond_tab)

    for l in range(NUM_LAYER):
        partials = _sc_msg(h, src_t, dst_t, code_t, tbl[l])
        mlp = _mlp_mid if l < NUM_LAYER - 1 else _mlp_last
        h = mlp(h, partials, W1[l], b1[l].reshape(1, -1), g1[l].reshape(1, -1),
                be1[l].reshape(1, -1), W2[l], b2[l].reshape(1, -1),
                bng[l].reshape(1, -1), bnb[l].reshape(1, -1),
                eps_p[l].reshape(1))
    return h


# speedup vs baseline: 2.2941x; 2.2941x over previous
import functools

import numpy as np
import jax
import jax.numpy as jnp
from jax import lax
from jax.experimental import pallas as pl
from jax.experimental.pallas import tpu as pltpu
from jax.experimental.pallas import tpu_sc as plsc

NUM_LAYER = 3
EMB = 128
HEMB = EMB // 2
N_NODES = 10000
N_EDGES = 320000
ATOM_FEATS = 9
ATOM_VOCAB = 120
BOND_FEATS = 3
BOND_VOCAB = 6

NC = 2
NS = 16
LANES = 16
NW = NC * NS

CHUNK = 128
CPW = -(-N_EDGES // (NW * CHUNK))
EPW = CPW * CHUNK
EP = NW * EPW

NRANGE = 2
RNG_ROWS = 5056
ACC_ROWS = 5120
RPT = ACC_ROWS // NS
DUMMY_R = RNG_ROWS
_TILE_CHUNKS = [(0, 128), (128, 128), (256, 64)]

NCODE = BOND_VOCAB ** BOND_FEATS

_MM = dict(preferred_element_type=jnp.float32, precision=lax.Precision.HIGHEST)


def _sc_msg_body(h_hbm, tbl_hbm, src_hbm, dst0_hbm, dst1_hbm, code_hbm,
                 out_hbm, src_v, dst0_v, dst1_v, code_v, hbuf, ebuf, acc_sh):
    c = lax.axis_index("c")
    s = lax.axis_index("s")
    wid = c * NS + s

    pltpu.sync_copy(src_hbm.at[wid], src_v)
    pltpu.sync_copy(dst0_hbm.at[wid], dst0_v)
    pltpu.sync_copy(dst1_hbm.at[wid], dst1_v)
    pltpu.sync_copy(code_hbm.at[wid], code_v)

    r0 = s * RPT
    for rng, dst_v in enumerate((dst0_v, dst1_v)):
        @pl.loop(0, CHUNK)
        def _zero(r):
            for k in range(EMB // LANES):
                hbuf[r, pl.ds(k * LANES, LANES)] = jnp.zeros((LANES,),
                                                             jnp.float32)

        for off, n in _TILE_CHUNKS:
            pltpu.sync_copy(hbuf.at[pl.ds(0, n)], acc_sh.at[pl.ds(r0 + off, n)])
        plsc.subcore_barrier()

        @pl.loop(0, CPW)
        def _chunk(j):
            pltpu.sync_copy(h_hbm.at[src_v.at[j]], hbuf)
            pltpu.sync_copy(tbl_hbm.at[code_v.at[j]], ebuf)

            @pl.loop(0, CHUNK)
            def _row(r):
                for k in range(EMB // LANES):
                    sl = (r, pl.ds(k * LANES, LANES))
                    hbuf[sl] = jnp.maximum(hbuf[sl] + ebuf[sl], 0.0)

            pltpu.sync_copy(hbuf, acc_sh.at[dst_v.at[j]], add=True)

        plsc.subcore_barrier()
        for off, n in _TILE_CHUNKS:
            pltpu.sync_copy(acc_sh.at[pl.ds(r0 + off, n)], hbuf.at[pl.ds(0, n)])
            pltpu.sync_copy(hbuf.at[pl.ds(0, n)],
                            out_hbm.at[c, rng, pl.ds(r0 + off, n)])
        plsc.subcore_barrier()


@functools.cache
def _get_sc_msg():
    return pl.kernel(
        _sc_msg_body,
        out_type=jax.ShapeDtypeStruct((NC, NRANGE, ACC_ROWS, EMB),
                                      jnp.float32),
        mesh=plsc.VectorSubcoreMesh(core_axis_name="c", subcore_axis_name="s",
                                    num_cores=NC, num_subcores=NS),
        scratch_types=[
            pltpu.VMEM((CPW, CHUNK), jnp.int32),
            pltpu.VMEM((CPW, CHUNK), jnp.int32),
            pltpu.VMEM((CPW, CHUNK), jnp.int32),
            pltpu.VMEM((CPW, CHUNK), jnp.int32),
            pltpu.VMEM((CHUNK, EMB), jnp.float32),
            pltpu.VMEM((CHUNK, EMB), jnp.float32),
            pltpu.VMEM_SHARED((ACC_ROWS, EMB), jnp.float32),
        ],
    )


_PREP_BLK = 1000


def _prep_body(x_ref, atab_ref, btab_ref, h_ref, t_ref):
    xv = x_ref[...]
    h = jnp.zeros((_PREP_BLK, EMB), jnp.float32)
    iota_a = lax.broadcasted_iota(jnp.int32, (1, ATOM_VOCAB), 1)
    for col in range(ATOM_FEATS):
        oh = (xv[:, col:col + 1] == iota_a).astype(jnp.float32)
        h = h + jnp.dot(oh, atab_ref[col], **_MM)
    h_ref[...] = h

    @pl.when(pl.program_id(0) == 0)
    def _():
        codes = lax.broadcasted_iota(jnp.int32, (NCODE, 1), 0)
        digs = [codes // 36, (codes // 6) % 6, codes % 6]
        iota_b = lax.broadcasted_iota(jnp.int32, (1, BOND_VOCAB), 1)
        for l in range(NUM_LAYER):
            t = jnp.zeros((NCODE, EMB), jnp.float32)
            for col in range(BOND_FEATS):
                oh = (digs[col] == iota_b).astype(jnp.float32)
                t = t + jnp.dot(oh, btab_ref[l, col], **_MM)
            t_ref[l] = t


_prep = pl.pallas_call(
    _prep_body,
    grid=(N_NODES // _PREP_BLK,),
    in_specs=[
        pl.BlockSpec((_PREP_BLK, ATOM_FEATS), lambda i: (i, 0)),
        pl.BlockSpec((ATOM_FEATS, ATOM_VOCAB, EMB), lambda i: (0, 0, 0)),
        pl.BlockSpec((NUM_LAYER, BOND_FEATS, BOND_VOCAB, EMB),
                     lambda i: (0, 0, 0, 0)),
    ],
    out_specs=[
        pl.BlockSpec((_PREP_BLK, EMB), lambda i: (i, 0)),
        pl.BlockSpec((NUM_LAYER, NCODE, EMB), lambda i: (0, 0, 0)),
    ],
    out_shape=[
        jax.ShapeDtypeStruct((N_NODES, EMB), jnp.float32),
        jax.ShapeDtypeStruct((NUM_LAYER, NCODE, EMB), jnp.float32),
    ],
)


def _bn(z, g, b):
    m = jnp.mean(z, axis=0, keepdims=True)
    zc = z - m
    v = jnp.mean(zc * zc, axis=0, keepdims=True)
    return zc * lax.rsqrt(v + 1e-5) * g + b


def _mlp_body(h_ref, p_ref, w1_ref, b1_ref, g1_ref, be1_ref, w2_ref,
              b2_ref, bng_ref, bnb_ref, eps_ref, o_ref, *, last):
    h = h_ref[...]
    aggr = jnp.concatenate(
        [p_ref[0, 0, :RNG_ROWS, :] + p_ref[1, 0, :RNG_ROWS, :],
         p_ref[0, 1, :N_NODES - RNG_ROWS, :]
         + p_ref[1, 1, :N_NODES - RNG_ROWS, :]], axis=0)
    z = (1.0 + eps_ref[0]) * h + aggr
    y = jnp.dot(z, w1_ref[...],
                preferred_element_type=jnp.float32) + b1_ref[...]
    y = _bn(y, g1_ref[...], be1_ref[...])
    y = jnp.maximum(y, 0.0)
    z2 = jnp.dot(y, w2_ref[...],
                 preferred_element_type=jnp.float32) + b2_ref[...]
    z2 = _bn(z2, bng_ref[...], bnb_ref[...])
    if not last:
        z2 = jnp.maximum(z2, 0.0)
    o_ref[...] = z2


def _make_mlp(last):
    return pl.pallas_call(
        functools.partial(_mlp_body, last=last),
        out_shape=jax.ShapeDtypeStruct((N_NODES, EMB), jnp.float32),
        in_specs=[pl.BlockSpec(memory_space=pltpu.VMEM)] * 10
        + [pl.BlockSpec(memory_space=pltpu.SMEM)],
    )


_mlp_mid = _make_mlp(False)
_mlp_last = _make_mlp(True)


def kernel(x, edge_index, edge_attr, atom_tab, bond_tab,
           W1, b1, g1, be1, W2, b2, eps_p, bng, bnb):
    x = x.astype(jnp.int32)
    src = edge_index[0].astype(jnp.int32)
    dst = edge_index[1].astype(jnp.int32)
    ea = edge_attr.astype(jnp.int32)
    code = ea[:, 0] * 36 + ea[:, 1] * 6 + ea[:, 2]

    pad = EP - N_EDGES
    src_t = jnp.concatenate([src, jnp.zeros((pad,), jnp.int32)])
    dst_t = jnp.concatenate([dst, jnp.full((pad,), NRANGE * RNG_ROWS,
                                           jnp.int32)])
    code_t = jnp.concatenate([code, jnp.zeros((pad,), jnp.int32)])
    dst0_t = jnp.where(dst_t < RNG_ROWS, dst_t, DUMMY_R)
    dst1_t = jnp.where(dst_t >= RNG_ROWS,
                       jnp.minimum(dst_t - RNG_ROWS, DUMMY_R), DUMMY_R)
    src_t = src_t.reshape(NW, CPW, CHUNK)
    dst0_t = dst0_t.reshape(NW, CPW, CHUNK)
    dst1_t = dst1_t.reshape(NW, CPW, CHUNK)
    code_t = code_t.reshape(NW, CPW, CHUNK)

    h, tbl = _prep(x, atom_tab, bond_tab)

    sc_msg = _get_sc_msg()
    for l in range(NUM_LAYER):
        partials = sc_msg(h, tbl[l], src_t, dst0_t, dst1_t, code_t)
        mlp = _mlp_mid if l < NUM_LAYER - 1 else _mlp_last
        h = mlp(h, partials, W1[l], b1[l].reshape(1, -1), g1[l].reshape(1, -1),
                be1[l].reshape(1, -1), W2[l], b2[l].reshape(1, -1),
                bng[l].reshape(1, -1), bnb[l].reshape(1, -1),
                eps_p[l].reshape(1))
    return h


# speedup vs baseline: 2.4319x; 1.0601x over previous
import functools

import numpy as np
import jax
import jax.numpy as jnp
from jax import lax
from jax.experimental import pallas as pl
from jax.experimental.pallas import tpu as pltpu
from jax.experimental.pallas import tpu_sc as plsc

NUM_LAYER = 3
EMB = 128
HEMB = EMB // 2
N_NODES = 10000
N_EDGES = 320000
ATOM_FEATS = 9
ATOM_VOCAB = 120
BOND_FEATS = 3
BOND_VOCAB = 6

NC = 2
NS = 16
LANES = 16
NW = NC * NS

CHUNK = 128
CPW = 2 * -(-N_EDGES // (NW * CHUNK * 2))
EPW = CPW * CHUNK
EP = NW * EPW

ACC_ROWS = 10112
RPT = ACC_ROWS // NS
DUMMY = N_NODES
_TILE_CHUNKS = [(0, 128), (128, 128), (256, 128), (384, 128), (512, 120)]

NCODE = BOND_VOCAB ** BOND_FEATS

_MM = dict(preferred_element_type=jnp.float32, precision=lax.Precision.HIGHEST)


def _sc_msg_body(h_hbm, tbl_hbm, idx_hbm, out_hbm,
                 sp_v, dp_v, cp_v, sidx, didx, eidx, hbuf, ebuf, acc_sh):
    c = lax.axis_index("c")
    s = lax.axis_index("s")
    wid = c * NS + s

    pltpu.sync_copy(idx_hbm.at[0, wid], sp_v)
    pltpu.sync_copy(idx_hbm.at[1, wid], dp_v)
    pltpu.sync_copy(idx_hbm.at[2, wid], cp_v)
    del wid

    @pl.loop(0, CHUNK)
    def _zero(r):
        for k in range(EMB // LANES):
            hbuf[r, pl.ds(k * LANES, LANES)] = jnp.zeros((LANES,), jnp.float32)

    r0 = s * RPT
    for off, n in _TILE_CHUNKS:
        pltpu.sync_copy(hbuf.at[pl.ds(0, n)], acc_sh.at[pl.ds(r0 + off, n)])
    plsc.subcore_barrier()

    @pl.loop(0, CPW // 2)
    def _group(j):
        @pl.loop(0, CHUNK // LANES)
        def _unpack(i):
            for packed, idx32 in ((sp_v, sidx), (dp_v, didx), (cp_v, eidx)):
                v = packed[j, pl.ds(i * LANES, LANES)]
                idx32[0, pl.ds(i * LANES, LANES)] = lax.bitwise_and(v, 0xFFFF)
                idx32[1, pl.ds(i * LANES, LANES)] = \
                    lax.shift_right_logical(v, 16)

        for sub in range(2):
            pltpu.sync_copy(h_hbm.at[sidx.at[sub]], hbuf)
            pltpu.sync_copy(tbl_hbm.at[eidx.at[sub]], ebuf)

            @pl.loop(0, CHUNK)
            def _row(r):
                for k in range(EMB // LANES):
                    sl = (r, pl.ds(k * LANES, LANES))
                    hbuf[sl] = jnp.maximum(hbuf[sl] + ebuf[sl], 0.0)

            pltpu.sync_copy(hbuf, acc_sh.at[didx.at[sub]], add=True)

    plsc.subcore_barrier()
    for off, n in _TILE_CHUNKS:
        pltpu.sync_copy(acc_sh.at[pl.ds(r0 + off, n)], hbuf.at[pl.ds(0, n)])
        pltpu.sync_copy(hbuf.at[pl.ds(0, n)],
                        out_hbm.at[c, pl.ds(r0 + off, n)])


@functools.cache
def _get_sc_msg():
    return pl.kernel(
        _sc_msg_body,
        out_type=jax.ShapeDtypeStruct((NC, ACC_ROWS, EMB), jnp.float32),
        mesh=plsc.VectorSubcoreMesh(core_axis_name="c", subcore_axis_name="s",
                                    num_cores=NC, num_subcores=NS),
        scratch_types=[
            pltpu.VMEM((CPW // 2, CHUNK), jnp.int32),
            pltpu.VMEM((CPW // 2, CHUNK), jnp.int32),
            pltpu.VMEM((CPW // 2, CHUNK), jnp.int32),
            pltpu.VMEM((2, CHUNK), jnp.int32),
            pltpu.VMEM((2, CHUNK), jnp.int32),
            pltpu.VMEM((2, CHUNK), jnp.int32),
            pltpu.VMEM((CHUNK, EMB), jnp.float32),
            pltpu.VMEM((CHUNK, EMB), jnp.float32),
            pltpu.VMEM_SHARED((ACC_ROWS, EMB), jnp.float32),
        ],
    )


_PREP_BLK = 1000


def _prep_body(x_ref, atab_ref, btab_ref, h_ref, t_ref):
    xv = x_ref[...]
    h = jnp.zeros((_PREP_BLK, EMB), jnp.float32)
    iota_a = lax.broadcasted_iota(jnp.int32, (1, ATOM_VOCAB), 1)
    for col in range(ATOM_FEATS):
        oh = (xv[:, col:col + 1] == iota_a).astype(jnp.float32)
        h = h + jnp.dot(oh, atab_ref[col], **_MM)
    h_ref[...] = h

    @pl.when(pl.program_id(0) == 0)
    def _():
        codes = lax.broadcasted_iota(jnp.int32, (NCODE, 1), 0)
        digs = [codes // 36, (codes // 6) % 6, codes % 6]
        iota_b = lax.broadcasted_iota(jnp.int32, (1, BOND_VOCAB), 1)
        for l in range(NUM_LAYER):
            t = jnp.zeros((NCODE, EMB), jnp.float32)
            for col in range(BOND_FEATS):
                oh = (digs[col] == iota_b).astype(jnp.float32)
                t = t + jnp.dot(oh, btab_ref[l, col], **_MM)
            t_ref[l] = t


_prep = pl.pallas_call(
    _prep_body,
    grid=(N_NODES // _PREP_BLK,),
    in_specs=[
        pl.BlockSpec((_PREP_BLK, ATOM_FEATS), lambda i: (i, 0)),
        pl.BlockSpec((ATOM_FEATS, ATOM_VOCAB, EMB), lambda i: (0, 0, 0)),
        pl.BlockSpec((NUM_LAYER, BOND_FEATS, BOND_VOCAB, EMB),
                     lambda i: (0, 0, 0, 0)),
    ],
    out_specs=[
        pl.BlockSpec((_PREP_BLK, EMB), lambda i: (i, 0)),
        pl.BlockSpec((NUM_LAYER, NCODE, EMB), lambda i: (0, 0, 0)),
    ],
    out_shape=[
        jax.ShapeDtypeStruct((N_NODES, EMB), jnp.float32),
        jax.ShapeDtypeStruct((NUM_LAYER, NCODE, EMB), jnp.float32),
    ],
)


def _bn(z, g, b):
    m = jnp.mean(z, axis=0, keepdims=True)
    zc = z - m
    v = jnp.mean(zc * zc, axis=0, keepdims=True)
    return zc * lax.rsqrt(v + 1e-5) * g + b


def _mlp_body(h_ref, p_ref, w1_ref, b1_ref, g1_ref, be1_ref, w2_ref,
              b2_ref, bng_ref, bnb_ref, eps_ref, o_ref, *, last):
    h = h_ref[...]
    aggr = p_ref[0, :N_NODES, :] + p_ref[1, :N_NODES, :]
    z = (1.0 + eps_ref[0]) * h + aggr
    y = jnp.dot(z, w1_ref[...],
                preferred_element_type=jnp.float32) + b1_ref[...]
    y = _bn(y, g1_ref[...], be1_ref[...])
    y = jnp.maximum(y, 0.0)
    z2 = jnp.dot(y, w2_ref[...],
                 preferred_element_type=jnp.float32) + b2_ref[...]
    z2 = _bn(z2, bng_ref[...], bnb_ref[...])
    if not last:
        z2 = jnp.maximum(z2, 0.0)
    o_ref[...] = z2


def _make_mlp(last):
    return pl.pallas_call(
        functools.partial(_mlp_body, last=last),
        out_shape=jax.ShapeDtypeStruct((N_NODES, EMB), jnp.float32),
        in_specs=[pl.BlockSpec(memory_space=pltpu.VMEM)] * 10
        + [pl.BlockSpec(memory_space=pltpu.SMEM)],
    )


_mlp_mid = _make_mlp(False)
_mlp_last = _make_mlp(True)


def kernel(x, edge_index, edge_attr, atom_tab, bond_tab,
           W1, b1, g1, be1, W2, b2, eps_p, bng, bnb):
    x = x.astype(jnp.int32)
    src = edge_index[0].astype(jnp.int32)
    dst = edge_index[1].astype(jnp.int32)
    ea = edge_attr.astype(jnp.int32)
    code = ea[:, 0] * 36 + ea[:, 1] * 6 + ea[:, 2]

    pad = EP - N_EDGES
    src_t = jnp.concatenate([src, jnp.zeros((pad,), jnp.int32)])
    dst_t = jnp.concatenate([dst, jnp.full((pad,), DUMMY, jnp.int32)])
    code_t = jnp.concatenate([code, jnp.zeros((pad,), jnp.int32)])
    idx3 = jnp.stack([src_t, dst_t, code_t]).reshape(3, NW, CPW // 2, CHUNK, 2)
    idx_t = idx3[..., 0] | (idx3[..., 1] << 16)

    h, tbl = _prep(x, atom_tab, bond_tab)

    sc_msg = _get_sc_msg()
    for l in range(NUM_LAYER):
        partials = sc_msg(h, tbl[l], idx_t)
        mlp = _mlp_mid if l < NUM_LAYER - 1 else _mlp_last
        h = mlp(h, partials, W1[l], b1[l].reshape(1, -1), g1[l].reshape(1, -1),
                be1[l].reshape(1, -1), W2[l], b2[l].reshape(1, -1),
                bng[l].reshape(1, -1), bnb[l].reshape(1, -1),
                eps_p[l].reshape(1))
    return h
